# Initial kernel scaffold; baseline (speedup 1.0000x reference)
import jax
import jax.numpy as jnp
from jax.experimental import pallas as pl


def kernel(confidence_map, tokens):
    raise NotImplementedError("write your pallas kernel here")


# speedup vs baseline: 1.4648x; 1.4648x over previous
import functools

import jax
import jax.numpy as jnp
from jax import lax
from jax.experimental import pallas as pl
from jax.experimental.pallas import tpu as pltpu
from jax.experimental.pallas import tpu_sc as plsc

G = 32
KH = 16
B = 32
H = 512
W = 512
N = G * G
D = 768
THR = 0.6
L = 16
NC = 2
NS = 16
CH = 64
NCH = N // CH

_mesh = plsc.VectorSubcoreMesh(
    core_axis_name="c", subcore_axis_name="s", num_cores=NC, num_subcores=NS)


@functools.partial(
    pl.kernel,
    out_type=(
        jax.ShapeDtypeStruct((B * N,), jnp.float32),
        jax.ShapeDtypeStruct((B * N,), jnp.int32),
        jax.ShapeDtypeStruct((B * L,), jnp.int32),
        jax.ShapeDtypeStruct((B * N, D), jnp.float32),
    ),
    mesh=_mesh,
    scratch_types=[
        pltpu.VMEM((KH * W,), jnp.float32),
        pltpu.VMEM((W,), jnp.float32),
        pltpu.VMEM((N,), jnp.float32),
        pltpu.VMEM((N,), jnp.int32),
        pltpu.VMEM((N,), jnp.int32),
        pltpu.VMEM((N,), jnp.int32),
        pltpu.VMEM((L,), jnp.int32),
        pltpu.VMEM((CH, D), jnp.float32),
        pltpu.VMEM((CH, D), jnp.float32),
        pltpu.SemaphoreType.DMA,
        pltpu.SemaphoreType.DMA,
        pltpu.SemaphoreType.DMA,
        pltpu.SemaphoreType.DMA,
    ],
    compiler_params=pltpu.CompilerParams(needs_layout_passes=False),
)
def _router_kernel(conf_hbm, tok_hbm, cg_hbm, order_hbm, nk_hbm, out_hbm,
                   cbuf, rsum, cg_v, kbbuf, orderbuf, gidxbuf, nkv,
                   rbuf0, rbuf1, gsem0, gsem1, wsem0, wsem1):
    b = lax.axis_index("c") * NS + lax.axis_index("s")
    iota = lax.iota(jnp.int32, L)

    row_words = KH * W

    def pool_row(g, carry):
        pltpu.sync_copy(
            conf_hbm.at[pl.ds(b * (H * W) + g * row_words, row_words)], cbuf)
        for v in range(W // L):
            rsum[pl.ds(v * L, L)] = cbuf[pl.ds(v * L, L)]

        def vert(r, c2):
            for v in range(W // L):
                plsc.addupdate(rsum.at[pl.ds(v * L, L)],
                               cbuf[pl.ds(r * W + v * L, L)])
            return c2

        lax.fori_loop(1, KH, vert, 0)
        for half in range(2):
            s = jnp.zeros((L,), jnp.float32)
            base = half * (L * L)
            for k in range(L):
                s = s + plsc.load_gather(rsum, [base + iota * L + k])
            s = s * jnp.float32(1.0 / (KH * KH))
            plsc.store_scatter(cg_v, [g * G + half * L + iota], s)
        return carry

    lax.fori_loop(0, G, pool_row, 0)
    pltpu.sync_copy(cg_v, cg_hbm.at[pl.ds(b * N, N)])

    nk = jnp.int32(0)
    for c in range(N // L):
        v = cg_v[pl.ds(c * L, L)]
        keep = (v <= THR).astype(jnp.int32)
        incl = plsc.cumsum(keep)
        kbbuf[pl.ds(c * L, L)] = incl - keep + nk
        nk = nk + jnp.sum(keep)

    nkv[...] = jnp.full((L,), nk, jnp.int32)
    pltpu.sync_copy(nkv, nk_hbm.at[pl.ds(b * L, L)])

    for c in range(N // L):
        i_vec = jnp.int32(c * L) + iota
        v = cg_v[pl.ds(c * L, L)]
        kb = kbbuf[pl.ds(c * L, L)]
        dest = jnp.where(v > THR, nk + i_vec - kb, kb)
        plsc.store_scatter(orderbuf, [dest], i_vec)

    for c in range(N // L):
        gidxbuf[pl.ds(c * L, L)] = orderbuf[pl.ds(c * L, L)] + b * N

    pltpu.sync_copy(orderbuf, order_hbm.at[pl.ds(b * N, N)])

    rbufs = (rbuf0, rbuf1)
    gsems = (gsem0, gsem1)
    wsems = (wsem0, wsem1)

    def start_gather(c):
        return pltpu.async_copy(
            tok_hbm.at[gidxbuf.at[pl.ds(c * CH, CH)]], rbufs[c % 2],
            gsems[c % 2])

    def start_write(c):
        return pltpu.async_copy(
            rbufs[c % 2], out_hbm.at[pl.ds(b * N + c * CH, CH)], wsems[c % 2])

    gd = start_gather(0)
    wds = [None] * NCH
    for c in range(NCH):
        gd.wait()
        wds[c] = start_write(c)
        if c + 1 < NCH:
            if c >= 1:
                wds[c - 1].wait()
            gd = start_gather(c + 1)
    wds[NCH - 2].wait()
    wds[NCH - 1].wait()


def kernel(confidence_map, tokens):
    conf_flat = confidence_map.reshape(B * H * W)
    tok2 = tokens.reshape(B * N, D)
    cg, order, nk, st = _router_kernel(conf_flat, tok2)
    conf_grid = cg.reshape(B, G, G)
    order = order.reshape(B, N)
    num_keep = nk.reshape(B, L)[:, 0]
    sorted_tokens = st.reshape(B, N, D)
    prune_ratio = jnp.float32(1.0) - (
        num_keep.sum().astype(jnp.float32) / jnp.float32(B * N))
    return conf_grid, order, num_keep, sorted_tokens, prune_ratio


# speedup vs baseline: 2.2361x; 1.5266x over previous
import functools

import jax
import jax.numpy as jnp
from jax import lax
from jax.experimental import pallas as pl
from jax.experimental.pallas import tpu as pltpu
from jax.experimental.pallas import tpu_sc as plsc

G = 32
KH = 16
B = 32
H = 512
W = 512
N = G * G
D = 768
THR = 0.6
L = 16
NC = 2
NS = 16
CH = 32
NCHUNK = N // CH
NBUF = 4

_mesh = plsc.VectorSubcoreMesh(
    core_axis_name="c", subcore_axis_name="s", num_cores=NC, num_subcores=NS)


@functools.partial(
    pl.kernel,
    out_type=(
        jax.ShapeDtypeStruct((B * N,), jnp.float32),
        jax.ShapeDtypeStruct((B * N,), jnp.int32),
        jax.ShapeDtypeStruct((B * L,), jnp.int32),
        jax.ShapeDtypeStruct((B * N, D), jnp.float32),
    ),
    mesh=_mesh,
    scratch_types=[
        pltpu.VMEM((KH, W), jnp.float32),
        pltpu.VMEM((KH, W), jnp.float32),
        pltpu.VMEM((W,), jnp.float32),
        pltpu.VMEM((N,), jnp.float32),
        pltpu.VMEM((N,), jnp.int32),
        pltpu.VMEM((N,), jnp.int32),
        pltpu.VMEM((N,), jnp.int32),
        pltpu.VMEM((L,), jnp.int32),
        [pltpu.VMEM((CH, D), jnp.float32) for _ in range(NBUF)],
        [pltpu.VMEM((CH,), jnp.int32) for _ in range(NBUF)],
        [pltpu.SemaphoreType.DMA for _ in range(2)],
        [pltpu.SemaphoreType.DMA for _ in range(NBUF)],
        [pltpu.SemaphoreType.DMA for _ in range(NBUF)],
    ],
    compiler_params=pltpu.CompilerParams(needs_layout_passes=False),
)
def _router_kernel(conf_hbm, tok_hbm, cg_hbm, order_hbm, nk_hbm, out_hbm,
                   cbuf0, cbuf1, rsum, cg_v, kbbuf, orderbuf, gidxbuf, nkv,
                   rbufs, idxbufs, csems, gsems, wsems):
    b = lax.axis_index("c") * NS + lax.axis_index("s")
    iota = lax.iota(jnp.int32, L)
    cbufs = (cbuf0, cbuf1)

    def conf_dma(g, buf, sem):
        return pltpu.async_copy(
            conf_hbm.at[pl.ds(b * H + g * KH, KH)], buf, sem)

    def pool_compute(g, buf):
        for v in range(W // L):
            acc = buf[0, pl.ds(v * L, L)]
            for r in range(1, KH):
                acc = acc + buf[r, pl.ds(v * L, L)]
            rsum[pl.ds(v * L, L)] = acc
        for half in range(2):
            base = half * (L * L)
            s0 = plsc.load_gather(rsum, [base + iota * L + 0])
            s1 = plsc.load_gather(rsum, [base + iota * L + 1])
            s2 = plsc.load_gather(rsum, [base + iota * L + 2])
            s3 = plsc.load_gather(rsum, [base + iota * L + 3])
            for k in range(4, L, 4):
                s0 = s0 + plsc.load_gather(rsum, [base + iota * L + k])
                s1 = s1 + plsc.load_gather(rsum, [base + iota * L + k + 1])
                s2 = s2 + plsc.load_gather(rsum, [base + iota * L + k + 2])
                s3 = s3 + plsc.load_gather(rsum, [base + iota * L + k + 3])
            s = ((s0 + s1) + (s2 + s3)) * jnp.float32(1.0 / (KH * KH))
            plsc.store_scatter(cg_v, [g * G + half * L + iota], s)

    conf_dma(0, cbufs[0], csems[0])
    conf_dma(1, cbufs[1], csems[1])

    def pool_pair(g2, carry):
        g = g2 * 2
        pltpu.make_async_copy(
            conf_hbm.at[pl.ds(0, KH)], cbufs[0], csems[0]).wait()
        pool_compute(g, cbufs[0])

        @pl.when(g2 < G // 2 - 1)
        def _():
            conf_dma(g + 2, cbufs[0], csems[0])

        pltpu.make_async_copy(
            conf_hbm.at[pl.ds(0, KH)], cbufs[1], csems[1]).wait()
        pool_compute(g + 1, cbufs[1])

        @pl.when(g2 < G // 2 - 1)
        def _():
            conf_dma(g + 3, cbufs[1], csems[1])

        return carry

    lax.fori_loop(0, G // 2, pool_pair, 0)
    pltpu.sync_copy(cg_v, cg_hbm.at[pl.ds(b * N, N)])

    nk = jnp.int32(0)
    for c in range(N // L):
        v = cg_v[pl.ds(c * L, L)]
        keep = (v <= THR).astype(jnp.int32)
        incl = plsc.cumsum(keep)
        kbbuf[pl.ds(c * L, L)] = incl - keep + nk
        nk = nk + jnp.sum(keep)

    nkv[...] = jnp.full((L,), nk, jnp.int32)
    pltpu.sync_copy(nkv, nk_hbm.at[pl.ds(b * L, L)])

    for c in range(N // L):
        i_vec = jnp.int32(c * L) + iota
        v = cg_v[pl.ds(c * L, L)]
        kb = kbbuf[pl.ds(c * L, L)]
        dest = jnp.where(v > THR, nk + i_vec - kb, kb)
        plsc.store_scatter(orderbuf, [dest], i_vec)

    for c in range(N // L):
        gidxbuf[pl.ds(c * L, L)] = orderbuf[pl.ds(c * L, L)] + b * N

    pltpu.sync_copy(orderbuf, order_hbm.at[pl.ds(b * N, N)])

    def start_gather(c):
        ib = idxbufs[c % NBUF]
        for u in range(CH // L):
            ib[pl.ds(u * L, L)] = gidxbuf[pl.ds(c * CH + u * L, L)]
        return pltpu.async_copy(
            tok_hbm.at[ib], rbufs[c % NBUF], gsems[c % NBUF])

    def start_write(c):
        return pltpu.async_copy(
            rbufs[c % NBUF], out_hbm.at[pl.ds(b * N + c * CH, CH)],
            wsems[c % NBUF])

    gd = [None] * NCHUNK
    wd = [None] * NCHUNK
    for c in range(NBUF - 1):
        gd[c] = start_gather(c)
    for c in range(NCHUNK):
        gd[c].wait()
        wd[c] = start_write(c)
        n = c + NBUF - 1
        if n < NCHUNK:
            if c >= 1:
                wd[c - 1].wait()
            gd[n] = start_gather(n)
    for c in range(NCHUNK - NBUF, NCHUNK):
        wd[c].wait()


def kernel(confidence_map, tokens):
    conf2d = confidence_map.reshape(B * H, W)
    tok2 = tokens.reshape(B * N, D)
    cg, order, nk, st = _router_kernel(conf2d, tok2)
    conf_grid = cg.reshape(B, G, G)
    order = order.reshape(B, N)
    num_keep = nk.reshape(B, L)[:, 0]
    sorted_tokens = st.reshape(B, N, D)
    prune_ratio = jnp.float32(1.0) - (
        num_keep.sum().astype(jnp.float32) / jnp.float32(B * N))
    return conf_grid, order, num_keep, sorted_tokens, prune_ratio
